# baseline (speedup 1.0000x reference)
import dataclasses
import functools

import jax
import jax.numpy as jnp
from jax import lax
from jax.experimental import pallas as pl
from jax.experimental.pallas import tpu as pltpu
from jax.experimental.pallas import tpu_sc as plsc

NN = 10000
EE = 320000
FF = 128
HH = 64
CC = 40
NC, NS, L = 2, 16, 16
NW = NC * NS
NPAD = 10240
CHUNK = 80
EPW = EE // NW
NCHUNK = EPW // CHUNK
RB = 1000
ZPW = NPAD // NS

_mesh = plsc.VectorSubcoreMesh(core_axis_name="c", subcore_axis_name="s")
_cp = pltpu.CompilerParams()
if "needs_layout_passes" in pltpu.CompilerParams.__dataclass_fields__:
    _cp = dataclasses.replace(_cp, needs_layout_passes=False)
if "use_tc_tiling_on_sc" in pltpu.CompilerParams.__dataclass_fields__:
    _cp = dataclasses.replace(_cp, use_tc_tiling_on_sc=False)
_f32 = jnp.float32
_i32 = jnp.int32


def _ds(b, n):
    return pl.ds(b, n)


def _m1_body(x_ref, wg_ref, w1_ref, h_ref, xw_ref):
    x = x_ref[...]
    h_ref[...] = jnp.dot(x, wg_ref[...], preferred_element_type=_f32)
    xw_ref[...] = jnp.dot(x, w1_ref[...], preferred_element_type=_f32)


def _m1(inputs, W_gl, W1):
    return pl.pallas_call(
        _m1_body,
        grid=(NN // RB,),
        in_specs=[
            pl.BlockSpec((RB, FF), lambda i: (i, 0)),
            pl.BlockSpec((FF, HH), lambda i: (0, 0)),
            pl.BlockSpec((FF, HH), lambda i: (0, 0)),
        ],
        out_specs=[
            pl.BlockSpec((RB, HH), lambda i: (i, 0)),
            pl.BlockSpec((RB, HH), lambda i: (i, 0)),
        ],
        out_shape=[
            jax.ShapeDtypeStruct((NN, HH), _f32),
            jax.ShapeDtypeStruct((NN, HH), _f32),
        ],
    )(inputs, W_gl, W1)


@functools.partial(
    pl.kernel,
    out_type=(
        jax.ShapeDtypeStruct((EE,), _f32),
        jax.ShapeDtypeStruct((NW, NPAD), _f32),
    ),
    mesh=_mesh,
    compiler_params=_cp,
    scratch_types=[
        pltpu.VMEM((NPAD,), _f32),
        pltpu.VMEM((NCHUNK, CHUNK), _i32),
        pltpu.VMEM((NCHUNK, CHUNK), _i32),
        pltpu.VMEM((2 * CHUNK, HH), _f32),
        pltpu.VMEM((2 * CHUNK, HH), _f32),
        pltpu.VMEM((EPW,), _f32),
        pltpu.VMEM((HH * L,), _f32),
        pltpu.SemaphoreType.DMA,
    ],
)
def _ka(h_hbm, src_hbm, dst_hbm, at_hbm, ex_hbm, s_hbm,
        s_acc, isrc, idst, hs, hd, exall, at_v, sem):
    cid = lax.axis_index("c")
    sid = lax.axis_index("s")
    wid = cid * NS + sid
    lane = lax.iota(_i32, L)

    pltpu.sync_copy(at_hbm, at_v)
    pltpu.sync_copy(src_hbm.at[wid], isrc)
    pltpu.sync_copy(dst_hbm.at[wid], idst)

    zv = jnp.zeros((L,), _f32)

    @pl.loop(0, NPAD // L)
    def _(i):
        s_acc[_ds(i * L, L)] = zv

    pltpu.async_copy(h_hbm.at[isrc.at[0]], hs.at[_ds(0, CHUNK)], sem)
    pltpu.async_copy(h_hbm.at[idst.at[0]], hd.at[_ds(0, CHUNK)], sem)

    @pl.loop(0, NCHUNK)
    def _(c):
        boff = jnp.bitwise_and(c, 1) * CHUNK
        nboff = CHUNK - boff

        @pl.when(c + 1 < NCHUNK)
        def _():
            pltpu.async_copy(h_hbm.at[isrc.at[c + 1]],
                             hs.at[_ds(nboff, CHUNK)], sem)
            pltpu.async_copy(h_hbm.at[idst.at[c + 1]],
                             hd.at[_ds(nboff, CHUNK)], sem)

        pltpu.make_async_copy(h_hbm.at[isrc.at[c]],
                              hs.at[_ds(boff, CHUNK)], sem).wait()
        pltpu.make_async_copy(h_hbm.at[idst.at[c]],
                              hd.at[_ds(boff, CHUNK)], sem).wait()

        cv = jnp.full((L,), 0, _i32) + c
        for g in range(CHUNK // L):
            rowg = lane + (g * L)
            browg = rowg + boff
            accs = [jnp.zeros((L,), _f32) for _ in range(4)]
            for j in range(HH):
                colj = jnp.full((L,), j, _i32)
                hsj = plsc.load_gather(hs, [browg, colj])
                hdj = plsc.load_gather(hd, [browg, colj])
                aj = at_v[_ds(j * L, L)]
                accs[j % 4] = accs[j % 4] + aj * jnp.abs(hsj - hdj)
            acc = (accs[0] + accs[1]) + (accs[2] + accs[3])
            exv = jnp.exp(jnp.maximum(acc, 0.0))
            dstv = plsc.load_gather(idst, [cv, rowg])
            plsc.addupdate_scatter(s_acc, [dstv], exv)
            exall[_ds(c * CHUNK + g * L, L)] = exv

    pltpu.sync_copy(exall, ex_hbm.at[_ds(wid * EPW, EPW)])
    pltpu.sync_copy(s_acc, s_hbm.at[wid])


def _ms_body(sp_ref, rec_ref, ind_ref):
    s = jnp.sum(sp_ref[...], axis=0).reshape(1, NPAD)
    pos = s > 0.0
    rec_ref[...] = jnp.where(pos, 1.0 / s, 0.0)
    ind_ref[...] = jnp.where(pos, 1.0, 0.0)


def _ms(s_parts):
    return pl.pallas_call(
        _ms_body,
        grid=(1,),
        in_specs=[pl.BlockSpec((NW, NPAD), lambda i: (0, 0))],
        out_specs=[
            pl.BlockSpec((1, NPAD), lambda i: (0, 0)),
            pl.BlockSpec((1, NPAD), lambda i: (0, 0)),
        ],
        out_shape=[
            jax.ShapeDtypeStruct((1, NPAD), _f32),
            jax.ShapeDtypeStruct((1, NPAD), _f32),
        ],
    )(s_parts)


@functools.partial(
    pl.kernel,
    out_type=(
        jax.ShapeDtypeStruct((EE,), _f32),
        jax.ShapeDtypeStruct((EE,), _f32),
        jax.ShapeDtypeStruct((NC, NPAD, HH), _f32),
    ),
    mesh=_mesh,
    compiler_params=_cp,
    scratch_types=[
        pltpu.VMEM((NPAD,), _f32),
        pltpu.VMEM((NPAD,), _f32),
        pltpu.VMEM((NCHUNK, CHUNK), _i32),
        pltpu.VMEM((NCHUNK, CHUNK), _i32),
        pltpu.VMEM((EPW,), _f32),
        pltpu.VMEM((EPW,), _f32),
        pltpu.VMEM((EPW,), _f32),
        pltpu.VMEM((2 * CHUNK, HH), _f32),
        pltpu.VMEM_SHARED((NPAD, HH), _f32),
        pltpu.SemaphoreType.DMA,
        pltpu.SemaphoreType.DMA,
    ],
)
def _kb(rec_hbm, ind_hbm, ex_hbm, src_hbm, dst_hbm, xw_hbm, z_hbm,
        ew_hbm, nm_hbm, acc_hbm,
        rec, ind, isrc, idst, exall, ewall, nmall, xwb, acc_sh, sem_g, sem_a):
    cid = lax.axis_index("c")
    sid = lax.axis_index("s")
    wid = cid * NS + sid
    lane = lax.iota(_i32, L)

    pltpu.sync_copy(rec_hbm.at[0], rec)
    pltpu.sync_copy(ind_hbm.at[0], ind)
    pltpu.sync_copy(src_hbm.at[wid], isrc)
    pltpu.sync_copy(dst_hbm.at[wid], idst)
    pltpu.sync_copy(ex_hbm.at[_ds(wid * EPW, EPW)], exall)
    pltpu.sync_copy(z_hbm, acc_sh.at[_ds(sid * ZPW, ZPW)])
    plsc.subcore_barrier()

    @pl.loop(0, NCHUNK)
    def _(c):
        cv = jnp.full((L,), 0, _i32) + c
        for g in range(CHUNK // L):
            colg = lane + g * L
            srcv = plsc.load_gather(isrc, [cv, colg])
            dstv = plsc.load_gather(idst, [cv, colg])
            exv = exall[_ds(c * CHUNK + g * L, L)]
            ew = exv * plsc.load_gather(rec, [dstv])
            nm = ew * plsc.load_gather(ind, [srcv])
            ewall[_ds(c * CHUNK + g * L, L)] = ew
            nmall[_ds(c * CHUNK + g * L, L)] = nm

    pltpu.async_copy(xw_hbm.at[isrc.at[0]], xwb.at[_ds(0, CHUNK)], sem_g)

    @pl.loop(0, NCHUNK)
    def _(c):
        boff = jnp.bitwise_and(c, 1) * CHUNK
        nboff = CHUNK - boff

        @pl.when(c >= 1)
        def _():
            pltpu.make_async_copy(xwb.at[_ds(nboff, CHUNK)],
                                  acc_sh.at[idst.at[c - 1]], sem_a).wait()

        @pl.when(c + 1 < NCHUNK)
        def _():
            pltpu.async_copy(xw_hbm.at[isrc.at[c + 1]],
                             xwb.at[_ds(nboff, CHUNK)], sem_g)

        pltpu.make_async_copy(xw_hbm.at[isrc.at[c]],
                              xwb.at[_ds(boff, CHUNK)], sem_g).wait()

        @pl.loop(0, CHUNK // 4)
        def _(e4):
            e0 = e4 * 4
            for t in range(4):
                rowv = jnp.full((L,), 0, _i32) + (boff + e0 + t)
                nb = plsc.load_gather(nmall, [jnp.full((L,), 0, _i32)
                                              + (c * CHUNK + e0 + t)])
                for k in range(HH // L):
                    colv = lane + k * L
                    v = plsc.load_gather(xwb, [rowv, colv])
                    plsc.store_scatter(xwb, [rowv, colv], v * nb)

        pltpu.async_copy(xwb.at[_ds(boff, CHUNK)],
                         acc_sh.at[idst.at[c]], sem_a, add=True)

    pltpu.make_async_copy(xwb.at[_ds(0, CHUNK)],
                          acc_sh.at[idst.at[NCHUNK - 1]], sem_a).wait()
    pltpu.sync_copy(ewall, ew_hbm.at[_ds(wid * EPW, EPW)])
    pltpu.sync_copy(nmall, nm_hbm.at[_ds(wid * EPW, EPW)])

    plsc.subcore_barrier()
    pltpu.sync_copy(acc_sh.at[_ds(sid * ZPW, ZPW)],
                    acc_hbm.at[cid, _ds(sid * ZPW, ZPW)])


@functools.partial(
    pl.kernel,
    out_type=jax.ShapeDtypeStruct((NC, NPAD, HH), _f32),
    mesh=_mesh,
    compiler_params=_cp,
    scratch_types=[
        pltpu.VMEM((NCHUNK, CHUNK), _i32),
        pltpu.VMEM((NCHUNK, CHUNK), _i32),
        pltpu.VMEM((EPW,), _f32),
        pltpu.VMEM((2 * CHUNK, HH), _f32),
        pltpu.VMEM_SHARED((NPAD, HH), _f32),
        pltpu.SemaphoreType.DMA,
        pltpu.SemaphoreType.DMA,
    ],
)
def _kc(nm_hbm, src_hbm, dst_hbm, xw_hbm, z_hbm, acc_hbm,
        isrc, idst, nmall, xwb, acc_sh, sem_g, sem_a):
    cid = lax.axis_index("c")
    sid = lax.axis_index("s")
    wid = cid * NS + sid
    lane = lax.iota(_i32, L)

    pltpu.sync_copy(src_hbm.at[wid], isrc)
    pltpu.sync_copy(dst_hbm.at[wid], idst)
    pltpu.sync_copy(nm_hbm.at[_ds(wid * EPW, EPW)], nmall)
    pltpu.sync_copy(z_hbm, acc_sh.at[_ds(sid * ZPW, ZPW)])
    plsc.subcore_barrier()

    pltpu.async_copy(xw_hbm.at[isrc.at[0]], xwb.at[_ds(0, CHUNK)], sem_g)

    @pl.loop(0, NCHUNK)
    def _(c):
        boff = jnp.bitwise_and(c, 1) * CHUNK
        nboff = CHUNK - boff

        @pl.when(c >= 1)
        def _():
            pltpu.make_async_copy(xwb.at[_ds(nboff, CHUNK)],
                                  acc_sh.at[idst.at[c - 1]], sem_a).wait()

        @pl.when(c + 1 < NCHUNK)
        def _():
            pltpu.async_copy(xw_hbm.at[isrc.at[c + 1]],
                             xwb.at[_ds(nboff, CHUNK)], sem_g)

        pltpu.make_async_copy(xw_hbm.at[isrc.at[c]],
                              xwb.at[_ds(boff, CHUNK)], sem_g).wait()

        @pl.loop(0, CHUNK // 4)
        def _(e4):
            e0 = e4 * 4
            for t in range(4):
                rowv = jnp.full((L,), 0, _i32) + (boff + e0 + t)
                nb = plsc.load_gather(nmall, [jnp.full((L,), 0, _i32)
                                              + (c * CHUNK + e0 + t)])
                for k in range(HH // L):
                    colv = lane + k * L
                    v = plsc.load_gather(xwb, [rowv, colv])
                    plsc.store_scatter(xwb, [rowv, colv], v * nb)

        pltpu.async_copy(xwb.at[_ds(boff, CHUNK)],
                         acc_sh.at[idst.at[c]], sem_a, add=True)

    pltpu.make_async_copy(xwb.at[_ds(0, CHUNK)],
                          acc_sh.at[idst.at[NCHUNK - 1]], sem_a).wait()

    plsc.subcore_barrier()
    pltpu.sync_copy(acc_sh.at[_ds(sid * ZPW, ZPW)],
                    acc_hbm.at[cid, _ds(sid * ZPW, ZPW)])


def _m2_body(acc_ref, b1_ref, w2_ref, out_ref):
    a = acc_ref[0] + acc_ref[1]
    x1 = jnp.maximum(a + b1_ref[...], 0.0)
    out_ref[...] = jnp.dot(x1, w2_ref[...], preferred_element_type=_f32)


def _m2(acc1, b1, W2pad):
    return pl.pallas_call(
        _m2_body,
        grid=(NN // RB,),
        in_specs=[
            pl.BlockSpec((2, RB, HH), lambda i: (0, i, 0)),
            pl.BlockSpec((1, HH), lambda i: (0, 0)),
            pl.BlockSpec((HH, HH), lambda i: (0, 0)),
        ],
        out_specs=pl.BlockSpec((RB, HH), lambda i: (i, 0)),
        out_shape=jax.ShapeDtypeStruct((NN, HH), _f32),
    )(acc1, b1, W2pad)


def _m3_body(acc_ref, b2_ref, out_ref):
    out_ref[...] = acc_ref[0, :, :CC] + acc_ref[1, :, :CC] + b2_ref[...]


def _m3(acc2, b2):
    return pl.pallas_call(
        _m3_body,
        grid=(NN // RB,),
        in_specs=[
            pl.BlockSpec((2, RB, HH), lambda i: (0, i, 0)),
            pl.BlockSpec((1, CC), lambda i: (0, 0)),
        ],
        out_specs=pl.BlockSpec((RB, CC), lambda i: (i, 0)),
        out_shape=jax.ShapeDtypeStruct((NN, CC), _f32),
    )(acc2, b2)


def kernel(inputs, edge_index, W_gl, a_gl, W1, b1, W2, b2):
    src3 = edge_index[0].reshape(NW, NCHUNK, CHUNK)
    dst3 = edge_index[1].reshape(NW, NCHUNK, CHUNK)
    a_tiled = jnp.broadcast_to(a_gl[:, None], (HH, L)).reshape(HH * L)
    W2pad = jnp.zeros((HH, HH), _f32).at[:, :CC].set(W2)
    zblk = jnp.zeros((ZPW, HH), _f32)

    h, xw1 = _m1(inputs, W_gl, W1)
    ex, s_parts = _ka(h, src3, dst3, a_tiled)
    rec, ind = _ms(s_parts)
    ew, norm, acc1 = _kb(rec, ind, ex, src3, dst3, xw1, zblk)
    xw2 = _m2(acc1, b1.reshape(1, HH), W2pad)
    acc2 = _kc(norm, src3, dst3, xw2, zblk)
    x2 = _m3(acc2, b2.reshape(1, CC))
    return (h, ew, x2)


# speedup vs baseline: 1.0176x; 1.0176x over previous
import dataclasses
import functools

import jax
import jax.numpy as jnp
from jax import lax
from jax.experimental import pallas as pl
from jax.experimental.pallas import tpu as pltpu
from jax.experimental.pallas import tpu_sc as plsc

NN = 10000
EE = 320000
FF = 128
HH = 64
CC = 40
NC, NS, L = 2, 16, 16
NW = NC * NS
NPAD = 10240
CHUNK = 80
EPW = EE // NW
NCHUNK = EPW // CHUNK
RB = 1000
ZPW = NPAD // NS

_mesh = plsc.VectorSubcoreMesh(core_axis_name="c", subcore_axis_name="s")
_cp = pltpu.CompilerParams()
if "needs_layout_passes" in pltpu.CompilerParams.__dataclass_fields__:
    _cp = dataclasses.replace(_cp, needs_layout_passes=False)
if "use_tc_tiling_on_sc" in pltpu.CompilerParams.__dataclass_fields__:
    _cp = dataclasses.replace(_cp, use_tc_tiling_on_sc=False)
_f32 = jnp.float32
_i32 = jnp.int32


def _ds(b, n):
    return pl.ds(b, n)


def _m1_body(x_ref, wg_ref, w1_ref, h_ref, xw_ref):
    x = x_ref[...]
    h_ref[...] = jnp.dot(x, wg_ref[...], preferred_element_type=_f32)
    xw_ref[...] = jnp.dot(x, w1_ref[...], preferred_element_type=_f32)


def _m1(inputs, W_gl, W1):
    return pl.pallas_call(
        _m1_body,
        grid=(NN // RB,),
        in_specs=[
            pl.BlockSpec((RB, FF), lambda i: (i, 0)),
            pl.BlockSpec((FF, HH), lambda i: (0, 0)),
            pl.BlockSpec((FF, HH), lambda i: (0, 0)),
        ],
        out_specs=[
            pl.BlockSpec((RB, HH), lambda i: (i, 0)),
            pl.BlockSpec((RB, HH), lambda i: (i, 0)),
        ],
        out_shape=[
            jax.ShapeDtypeStruct((NN, HH), _f32),
            jax.ShapeDtypeStruct((NN, HH), _f32),
        ],
    )(inputs, W_gl, W1)


@functools.partial(
    pl.kernel,
    out_type=(
        jax.ShapeDtypeStruct((EE,), _f32),
        jax.ShapeDtypeStruct((NW, NPAD), _f32),
    ),
    mesh=_mesh,
    compiler_params=_cp,
    scratch_types=[
        pltpu.VMEM((NPAD,), _f32),
        pltpu.VMEM((NCHUNK, CHUNK), _i32),
        pltpu.VMEM((NCHUNK, CHUNK), _i32),
        pltpu.VMEM((2 * CHUNK, HH), _f32),
        pltpu.VMEM((2 * CHUNK, HH), _f32),
        pltpu.VMEM((EPW,), _f32),
        pltpu.VMEM((HH * L,), _f32),
        pltpu.SemaphoreType.DMA,
    ],
)
def _ka(h_hbm, src_hbm, dst_hbm, at_hbm, ex_hbm, s_hbm,
        s_acc, isrc, idst, hs, hd, exall, at_v, sem):
    cid = lax.axis_index("c")
    sid = lax.axis_index("s")
    wid = cid * NS + sid
    lane = lax.iota(_i32, L)

    pltpu.sync_copy(at_hbm, at_v)
    pltpu.sync_copy(src_hbm.at[wid], isrc)
    pltpu.sync_copy(dst_hbm.at[wid], idst)

    zv = jnp.zeros((L,), _f32)

    @pl.loop(0, NPAD // L)
    def _(i):
        s_acc[_ds(i * L, L)] = zv

    pltpu.async_copy(h_hbm.at[isrc.at[0]], hs.at[_ds(0, CHUNK)], sem)
    pltpu.async_copy(h_hbm.at[idst.at[0]], hd.at[_ds(0, CHUNK)], sem)

    @pl.loop(0, NCHUNK)
    def _(c):
        boff = jnp.bitwise_and(c, 1) * CHUNK
        nboff = CHUNK - boff

        @pl.when(c + 1 < NCHUNK)
        def _():
            pltpu.async_copy(h_hbm.at[isrc.at[c + 1]],
                             hs.at[_ds(nboff, CHUNK)], sem)
            pltpu.async_copy(h_hbm.at[idst.at[c + 1]],
                             hd.at[_ds(nboff, CHUNK)], sem)

        pltpu.make_async_copy(h_hbm.at[isrc.at[c]],
                              hs.at[_ds(boff, CHUNK)], sem).wait()
        pltpu.make_async_copy(h_hbm.at[idst.at[c]],
                              hd.at[_ds(boff, CHUNK)], sem).wait()

        cv = jnp.full((L,), 0, _i32) + c

        @pl.loop(0, CHUNK // L)
        def _(g):
            rowg = lane + g * L
            browg = rowg + boff
            accs = [jnp.zeros((L,), _f32) for _ in range(4)]
            for j in range(HH):
                colj = jnp.full((L,), j, _i32)
                hsj = plsc.load_gather(hs, [browg, colj])
                hdj = plsc.load_gather(hd, [browg, colj])
                aj = at_v[_ds(j * L, L)]
                accs[j % 4] = accs[j % 4] + aj * jnp.abs(hsj - hdj)
            acc = (accs[0] + accs[1]) + (accs[2] + accs[3])
            exv = jnp.exp(jnp.maximum(acc, 0.0))
            dstv = plsc.load_gather(idst, [cv, rowg])
            plsc.addupdate_scatter(s_acc, [dstv], exv)
            exall[_ds(c * CHUNK + g * L, L)] = exv

    pltpu.sync_copy(exall, ex_hbm.at[_ds(wid * EPW, EPW)])
    pltpu.sync_copy(s_acc, s_hbm.at[wid])


def _ms_body(sp_ref, rec_ref, ind_ref):
    s = jnp.sum(sp_ref[...], axis=0).reshape(1, NPAD)
    pos = s > 0.0
    rec_ref[...] = jnp.where(pos, 1.0 / s, 0.0)
    ind_ref[...] = jnp.where(pos, 1.0, 0.0)


def _ms(s_parts):
    return pl.pallas_call(
        _ms_body,
        grid=(1,),
        in_specs=[pl.BlockSpec((NW, NPAD), lambda i: (0, 0))],
        out_specs=[
            pl.BlockSpec((1, NPAD), lambda i: (0, 0)),
            pl.BlockSpec((1, NPAD), lambda i: (0, 0)),
        ],
        out_shape=[
            jax.ShapeDtypeStruct((1, NPAD), _f32),
            jax.ShapeDtypeStruct((1, NPAD), _f32),
        ],
    )(s_parts)


@functools.partial(
    pl.kernel,
    out_type=(
        jax.ShapeDtypeStruct((EE,), _f32),
        jax.ShapeDtypeStruct((EE,), _f32),
        jax.ShapeDtypeStruct((NC, NPAD, HH), _f32),
    ),
    mesh=_mesh,
    compiler_params=_cp,
    scratch_types=[
        pltpu.VMEM((NPAD,), _f32),
        pltpu.VMEM((NPAD,), _f32),
        pltpu.VMEM((NCHUNK, CHUNK), _i32),
        pltpu.VMEM((NCHUNK, CHUNK), _i32),
        pltpu.VMEM((EPW,), _f32),
        pltpu.VMEM((EPW,), _f32),
        pltpu.VMEM((EPW,), _f32),
        pltpu.VMEM((2 * CHUNK, HH), _f32),
        pltpu.VMEM_SHARED((NPAD, HH), _f32),
        pltpu.SemaphoreType.DMA,
        pltpu.SemaphoreType.DMA,
    ],
)
def _kb(rec_hbm, ind_hbm, ex_hbm, src_hbm, dst_hbm, xw_hbm, z_hbm,
        ew_hbm, nm_hbm, acc_hbm,
        rec, ind, isrc, idst, exall, ewall, nmall, xwb, acc_sh, sem_g, sem_a):
    cid = lax.axis_index("c")
    sid = lax.axis_index("s")
    wid = cid * NS + sid
    lane = lax.iota(_i32, L)

    pltpu.sync_copy(rec_hbm.at[0], rec)
    pltpu.sync_copy(ind_hbm.at[0], ind)
    pltpu.sync_copy(src_hbm.at[wid], isrc)
    pltpu.sync_copy(dst_hbm.at[wid], idst)
    pltpu.sync_copy(ex_hbm.at[_ds(wid * EPW, EPW)], exall)
    pltpu.sync_copy(z_hbm, acc_sh.at[_ds(sid * ZPW, ZPW)])
    plsc.subcore_barrier()

    @pl.loop(0, NCHUNK)
    def _(c):
        cv = jnp.full((L,), 0, _i32) + c
        for g in range(CHUNK // L):
            colg = lane + g * L
            srcv = plsc.load_gather(isrc, [cv, colg])
            dstv = plsc.load_gather(idst, [cv, colg])
            exv = exall[_ds(c * CHUNK + g * L, L)]
            ew = exv * plsc.load_gather(rec, [dstv])
            nm = ew * plsc.load_gather(ind, [srcv])
            ewall[_ds(c * CHUNK + g * L, L)] = ew
            nmall[_ds(c * CHUNK + g * L, L)] = nm

    pltpu.async_copy(xw_hbm.at[isrc.at[0]], xwb.at[_ds(0, CHUNK)], sem_g)

    @pl.loop(0, NCHUNK)
    def _(c):
        boff = jnp.bitwise_and(c, 1) * CHUNK
        nboff = CHUNK - boff

        @pl.when(c >= 1)
        def _():
            pltpu.make_async_copy(xwb.at[_ds(nboff, CHUNK)],
                                  acc_sh.at[idst.at[c - 1]], sem_a).wait()

        @pl.when(c + 1 < NCHUNK)
        def _():
            pltpu.async_copy(xw_hbm.at[isrc.at[c + 1]],
                             xwb.at[_ds(nboff, CHUNK)], sem_g)

        pltpu.make_async_copy(xw_hbm.at[isrc.at[c]],
                              xwb.at[_ds(boff, CHUNK)], sem_g).wait()

        @pl.loop(0, CHUNK // 4)
        def _(e4):
            e0 = e4 * 4
            for t in range(4):
                rowv = jnp.full((L,), 0, _i32) + (boff + e0 + t)
                nb = plsc.load_gather(nmall, [jnp.full((L,), 0, _i32)
                                              + (c * CHUNK + e0 + t)])
                for k in range(HH // L):
                    colv = lane + k * L
                    v = plsc.load_gather(xwb, [rowv, colv])
                    plsc.store_scatter(xwb, [rowv, colv], v * nb)

        pltpu.async_copy(xwb.at[_ds(boff, CHUNK)],
                         acc_sh.at[idst.at[c]], sem_a, add=True)

    pltpu.make_async_copy(xwb.at[_ds(0, CHUNK)],
                          acc_sh.at[idst.at[NCHUNK - 1]], sem_a).wait()
    pltpu.sync_copy(ewall, ew_hbm.at[_ds(wid * EPW, EPW)])
    pltpu.sync_copy(nmall, nm_hbm.at[_ds(wid * EPW, EPW)])

    plsc.subcore_barrier()
    pltpu.sync_copy(acc_sh.at[_ds(sid * ZPW, ZPW)],
                    acc_hbm.at[cid, _ds(sid * ZPW, ZPW)])


@functools.partial(
    pl.kernel,
    out_type=jax.ShapeDtypeStruct((NC, NPAD, HH), _f32),
    mesh=_mesh,
    compiler_params=_cp,
    scratch_types=[
        pltpu.VMEM((NCHUNK, CHUNK), _i32),
        pltpu.VMEM((NCHUNK, CHUNK), _i32),
        pltpu.VMEM((EPW,), _f32),
        pltpu.VMEM((2 * CHUNK, HH), _f32),
        pltpu.VMEM_SHARED((NPAD, HH), _f32),
        pltpu.SemaphoreType.DMA,
        pltpu.SemaphoreType.DMA,
    ],
)
def _kc(nm_hbm, src_hbm, dst_hbm, xw_hbm, z_hbm, acc_hbm,
        isrc, idst, nmall, xwb, acc_sh, sem_g, sem_a):
    cid = lax.axis_index("c")
    sid = lax.axis_index("s")
    wid = cid * NS + sid
    lane = lax.iota(_i32, L)

    pltpu.sync_copy(src_hbm.at[wid], isrc)
    pltpu.sync_copy(dst_hbm.at[wid], idst)
    pltpu.sync_copy(nm_hbm.at[_ds(wid * EPW, EPW)], nmall)
    pltpu.sync_copy(z_hbm, acc_sh.at[_ds(sid * ZPW, ZPW)])
    plsc.subcore_barrier()

    pltpu.async_copy(xw_hbm.at[isrc.at[0]], xwb.at[_ds(0, CHUNK)], sem_g)

    @pl.loop(0, NCHUNK)
    def _(c):
        boff = jnp.bitwise_and(c, 1) * CHUNK
        nboff = CHUNK - boff

        @pl.when(c >= 1)
        def _():
            pltpu.make_async_copy(xwb.at[_ds(nboff, CHUNK)],
                                  acc_sh.at[idst.at[c - 1]], sem_a).wait()

        @pl.when(c + 1 < NCHUNK)
        def _():
            pltpu.async_copy(xw_hbm.at[isrc.at[c + 1]],
                             xwb.at[_ds(nboff, CHUNK)], sem_g)

        pltpu.make_async_copy(xw_hbm.at[isrc.at[c]],
                              xwb.at[_ds(boff, CHUNK)], sem_g).wait()

        @pl.loop(0, CHUNK // 4)
        def _(e4):
            e0 = e4 * 4
            for t in range(4):
                rowv = jnp.full((L,), 0, _i32) + (boff + e0 + t)
                nb = plsc.load_gather(nmall, [jnp.full((L,), 0, _i32)
                                              + (c * CHUNK + e0 + t)])
                for k in range(HH // L):
                    colv = lane + k * L
                    v = plsc.load_gather(xwb, [rowv, colv])
                    plsc.store_scatter(xwb, [rowv, colv], v * nb)

        pltpu.async_copy(xwb.at[_ds(boff, CHUNK)],
                         acc_sh.at[idst.at[c]], sem_a, add=True)

    pltpu.make_async_copy(xwb.at[_ds(0, CHUNK)],
                          acc_sh.at[idst.at[NCHUNK - 1]], sem_a).wait()

    plsc.subcore_barrier()
    pltpu.sync_copy(acc_sh.at[_ds(sid * ZPW, ZPW)],
                    acc_hbm.at[cid, _ds(sid * ZPW, ZPW)])


def _m2_body(acc_ref, b1_ref, w2_ref, out_ref):
    a = acc_ref[0] + acc_ref[1]
    x1 = jnp.maximum(a + b1_ref[...], 0.0)
    out_ref[...] = jnp.dot(x1, w2_ref[...], preferred_element_type=_f32)


def _m2(acc1, b1, W2pad):
    return pl.pallas_call(
        _m2_body,
        grid=(NN // RB,),
        in_specs=[
            pl.BlockSpec((2, RB, HH), lambda i: (0, i, 0)),
            pl.BlockSpec((1, HH), lambda i: (0, 0)),
            pl.BlockSpec((HH, HH), lambda i: (0, 0)),
        ],
        out_specs=pl.BlockSpec((RB, HH), lambda i: (i, 0)),
        out_shape=jax.ShapeDtypeStruct((NN, HH), _f32),
    )(acc1, b1, W2pad)


def _m3_body(acc_ref, b2_ref, out_ref):
    out_ref[...] = acc_ref[0, :, :CC] + acc_ref[1, :, :CC] + b2_ref[...]


def _m3(acc2, b2):
    return pl.pallas_call(
        _m3_body,
        grid=(NN // RB,),
        in_specs=[
            pl.BlockSpec((2, RB, HH), lambda i: (0, i, 0)),
            pl.BlockSpec((1, CC), lambda i: (0, 0)),
        ],
        out_specs=pl.BlockSpec((RB, CC), lambda i: (i, 0)),
        out_shape=jax.ShapeDtypeStruct((NN, CC), _f32),
    )(acc2, b2)


def kernel(inputs, edge_index, W_gl, a_gl, W1, b1, W2, b2):
    src3 = edge_index[0].reshape(NW, NCHUNK, CHUNK)
    dst3 = edge_index[1].reshape(NW, NCHUNK, CHUNK)
    a_tiled = jnp.broadcast_to(a_gl[:, None], (HH, L)).reshape(HH * L)
    W2pad = jnp.zeros((HH, HH), _f32).at[:, :CC].set(W2)
    zblk = jnp.zeros((ZPW, HH), _f32)

    h, xw1 = _m1(inputs, W_gl, W1)
    ex, s_parts = _ka(h, src3, dst3, a_tiled)
    rec, ind = _ms(s_parts)
    ew, norm, acc1 = _kb(rec, ind, ex, src3, dst3, xw1, zblk)
    xw2 = _m2(acc1, b1.reshape(1, HH), W2pad)
    acc2 = _kc(norm, src3, dst3, xw2, zblk)
    x2 = _m3(acc2, b2.reshape(1, CC))
    return (h, ew, x2)


# speedup vs baseline: 1.6170x; 1.5891x over previous
import dataclasses
import functools

import jax
import jax.numpy as jnp
from jax import lax
from jax.experimental import pallas as pl
from jax.experimental.pallas import tpu as pltpu
from jax.experimental.pallas import tpu_sc as plsc

NN = 10000
EE = 320000
FF = 128
HH = 64
CC = 40
NC, NS, L = 2, 16, 16
NW = NC * NS
NPAD = 10240
CHUNK = 80
EPW = EE // NW
NCHUNK = EPW // CHUNK
RB = 1000
ZPW = NPAD // NS

_mesh = plsc.VectorSubcoreMesh(core_axis_name="c", subcore_axis_name="s")
_cp = pltpu.CompilerParams()
if "needs_layout_passes" in pltpu.CompilerParams.__dataclass_fields__:
    _cp = dataclasses.replace(_cp, needs_layout_passes=False)
if "use_tc_tiling_on_sc" in pltpu.CompilerParams.__dataclass_fields__:
    _cp = dataclasses.replace(_cp, use_tc_tiling_on_sc=False)
_f32 = jnp.float32
_i32 = jnp.int32


def _ds(b, n):
    return pl.ds(b, n)


def _m1_body(x_ref, wg_ref, w1_ref, h_ref, xw_ref):
    x = x_ref[...]
    h_ref[...] = jnp.dot(x, wg_ref[...], preferred_element_type=_f32)
    xw_ref[...] = jnp.dot(x, w1_ref[...], preferred_element_type=_f32)


def _m1(inputs, W_gl, W1):
    return pl.pallas_call(
        _m1_body,
        grid=(NN // RB,),
        in_specs=[
            pl.BlockSpec((RB, FF), lambda i: (i, 0)),
            pl.BlockSpec((FF, HH), lambda i: (0, 0)),
            pl.BlockSpec((FF, HH), lambda i: (0, 0)),
        ],
        out_specs=[
            pl.BlockSpec((RB, HH), lambda i: (i, 0)),
            pl.BlockSpec((RB, HH), lambda i: (i, 0)),
        ],
        out_shape=[
            jax.ShapeDtypeStruct((NN, HH), _f32),
            jax.ShapeDtypeStruct((NN, HH), _f32),
        ],
    )(inputs, W_gl, W1)


@functools.partial(
    pl.kernel,
    out_type=(
        jax.ShapeDtypeStruct((EE,), _f32),
        jax.ShapeDtypeStruct((NW, NPAD), _f32),
    ),
    mesh=_mesh,
    compiler_params=_cp,
    scratch_types=[
        pltpu.VMEM((NPAD,), _f32),
        pltpu.VMEM((NCHUNK, CHUNK), _i32),
        pltpu.VMEM((NCHUNK, CHUNK), _i32),
        pltpu.VMEM((2 * CHUNK, HH), _f32),
        pltpu.VMEM((2 * CHUNK, HH), _f32),
        pltpu.VMEM((EPW,), _f32),
        pltpu.VMEM((HH * L,), _f32),
        pltpu.SemaphoreType.DMA,
    ],
)
def _ka(h_hbm, src_hbm, dst_hbm, at_hbm, ex_hbm, s_hbm,
        s_acc, isrc, idst, hs, hd, exall, at_v, sem):
    cid = lax.axis_index("c")
    sid = lax.axis_index("s")
    wid = cid * NS + sid
    lane = lax.iota(_i32, L)

    pltpu.sync_copy(at_hbm, at_v)
    pltpu.sync_copy(src_hbm.at[wid], isrc)
    pltpu.sync_copy(dst_hbm.at[wid], idst)

    zv = jnp.zeros((L,), _f32)

    @pl.loop(0, NPAD // L)
    def _(i):
        s_acc[_ds(i * L, L)] = zv

    pltpu.async_copy(h_hbm.at[isrc.at[0]], hs.at[_ds(0, CHUNK)], sem)
    pltpu.async_copy(h_hbm.at[idst.at[0]], hd.at[_ds(0, CHUNK)], sem)

    @pl.loop(0, NCHUNK)
    def _(c):
        boff = jnp.bitwise_and(c, 1) * CHUNK
        nboff = CHUNK - boff

        @pl.when(c + 1 < NCHUNK)
        def _():
            pltpu.async_copy(h_hbm.at[isrc.at[c + 1]],
                             hs.at[_ds(nboff, CHUNK)], sem)
            pltpu.async_copy(h_hbm.at[idst.at[c + 1]],
                             hd.at[_ds(nboff, CHUNK)], sem)

        pltpu.make_async_copy(h_hbm.at[isrc.at[c]],
                              hs.at[_ds(boff, CHUNK)], sem).wait()
        pltpu.make_async_copy(h_hbm.at[idst.at[c]],
                              hd.at[_ds(boff, CHUNK)], sem).wait()

        cv = jnp.full((L,), 0, _i32) + c

        @pl.loop(0, CHUNK // L)
        def _(g):
            rowg = lane + g * L
            browg = rowg + boff
            accs = [jnp.zeros((L,), _f32) for _ in range(4)]
            for j in range(HH):
                colj = jnp.bitwise_and(lane + j, HH - 1)
                hsj = plsc.load_gather(hs, [browg, colj])
                hdj = plsc.load_gather(hd, [browg, colj])
                aj = at_v[_ds(j * L, L)]
                accs[j % 4] = accs[j % 4] + aj * jnp.abs(hsj - hdj)
            acc = (accs[0] + accs[1]) + (accs[2] + accs[3])
            exv = jnp.exp(jnp.maximum(acc, 0.0))
            dstv = plsc.load_gather(idst, [cv, rowg])
            plsc.addupdate_scatter(s_acc, [dstv], exv)
            exall[_ds(c * CHUNK + g * L, L)] = exv

    pltpu.sync_copy(exall, ex_hbm.at[_ds(wid * EPW, EPW)])
    pltpu.sync_copy(s_acc, s_hbm.at[wid])


def _ms_body(sp_ref, rec_ref, ind_ref):
    s = jnp.sum(sp_ref[...], axis=0).reshape(1, NPAD)
    pos = s > 0.0
    rec_ref[...] = jnp.where(pos, 1.0 / s, 0.0)
    ind_ref[...] = jnp.where(pos, 1.0, 0.0)


def _ms(s_parts):
    return pl.pallas_call(
        _ms_body,
        grid=(1,),
        in_specs=[pl.BlockSpec((NW, NPAD), lambda i: (0, 0))],
        out_specs=[
            pl.BlockSpec((1, NPAD), lambda i: (0, 0)),
            pl.BlockSpec((1, NPAD), lambda i: (0, 0)),
        ],
        out_shape=[
            jax.ShapeDtypeStruct((1, NPAD), _f32),
            jax.ShapeDtypeStruct((1, NPAD), _f32),
        ],
    )(s_parts)


@functools.partial(
    pl.kernel,
    out_type=(
        jax.ShapeDtypeStruct((EE,), _f32),
        jax.ShapeDtypeStruct((EE,), _f32),
        jax.ShapeDtypeStruct((NC, NPAD, HH), _f32),
    ),
    mesh=_mesh,
    compiler_params=_cp,
    scratch_types=[
        pltpu.VMEM((NPAD,), _f32),
        pltpu.VMEM((NPAD,), _f32),
        pltpu.VMEM((NCHUNK, CHUNK), _i32),
        pltpu.VMEM((NCHUNK, CHUNK), _i32),
        pltpu.VMEM((EPW,), _f32),
        pltpu.VMEM((EPW,), _f32),
        pltpu.VMEM((EPW,), _f32),
        pltpu.VMEM((2 * CHUNK, HH), _f32),
        pltpu.VMEM_SHARED((NPAD, HH), _f32),
        pltpu.SemaphoreType.DMA,
        pltpu.SemaphoreType.DMA,
    ],
)
def _kb(rec_hbm, ind_hbm, ex_hbm, src_hbm, dst_hbm, xw_hbm, z_hbm,
        ew_hbm, nm_hbm, acc_hbm,
        rec, ind, isrc, idst, exall, ewall, nmall, xwb, acc_sh, sem_g, sem_a):
    cid = lax.axis_index("c")
    sid = lax.axis_index("s")
    wid = cid * NS + sid
    lane = lax.iota(_i32, L)

    pltpu.sync_copy(rec_hbm.at[0], rec)
    pltpu.sync_copy(ind_hbm.at[0], ind)
    pltpu.sync_copy(src_hbm.at[wid], isrc)
    pltpu.sync_copy(dst_hbm.at[wid], idst)
    pltpu.sync_copy(ex_hbm.at[_ds(wid * EPW, EPW)], exall)
    pltpu.sync_copy(z_hbm, acc_sh.at[_ds(sid * ZPW, ZPW)])
    plsc.subcore_barrier()

    @pl.loop(0, NCHUNK)
    def _(c):
        cv = jnp.full((L,), 0, _i32) + c
        for g in range(CHUNK // L):
            colg = lane + g * L
            srcv = plsc.load_gather(isrc, [cv, colg])
            dstv = plsc.load_gather(idst, [cv, colg])
            exv = exall[_ds(c * CHUNK + g * L, L)]
            ew = exv * plsc.load_gather(rec, [dstv])
            nm = ew * plsc.load_gather(ind, [srcv])
            ewall[_ds(c * CHUNK + g * L, L)] = ew
            nmall[_ds(c * CHUNK + g * L, L)] = nm

    pltpu.async_copy(xw_hbm.at[isrc.at[0]], xwb.at[_ds(0, CHUNK)], sem_g)

    @pl.loop(0, NCHUNK)
    def _(c):
        boff = jnp.bitwise_and(c, 1) * CHUNK
        nboff = CHUNK - boff

        @pl.when(c >= 1)
        def _():
            pltpu.make_async_copy(xwb.at[_ds(nboff, CHUNK)],
                                  acc_sh.at[idst.at[c - 1]], sem_a).wait()

        @pl.when(c + 1 < NCHUNK)
        def _():
            pltpu.async_copy(xw_hbm.at[isrc.at[c + 1]],
                             xwb.at[_ds(nboff, CHUNK)], sem_g)

        pltpu.make_async_copy(xw_hbm.at[isrc.at[c]],
                              xwb.at[_ds(boff, CHUNK)], sem_g).wait()

        @pl.loop(0, CHUNK // 4)
        def _(e4):
            e0 = e4 * 4
            for t in range(4):
                rowv = jnp.full((L,), 0, _i32) + (boff + e0 + t)
                nb = plsc.load_gather(nmall, [jnp.full((L,), 0, _i32)
                                              + (c * CHUNK + e0 + t)])
                for k in range(HH // L):
                    colv = lane + k * L
                    v = plsc.load_gather(xwb, [rowv, colv])
                    plsc.store_scatter(xwb, [rowv, colv], v * nb)

        pltpu.async_copy(xwb.at[_ds(boff, CHUNK)],
                         acc_sh.at[idst.at[c]], sem_a, add=True)

    pltpu.make_async_copy(xwb.at[_ds(0, CHUNK)],
                          acc_sh.at[idst.at[NCHUNK - 1]], sem_a).wait()
    pltpu.sync_copy(ewall, ew_hbm.at[_ds(wid * EPW, EPW)])
    pltpu.sync_copy(nmall, nm_hbm.at[_ds(wid * EPW, EPW)])

    plsc.subcore_barrier()
    pltpu.sync_copy(acc_sh.at[_ds(sid * ZPW, ZPW)],
                    acc_hbm.at[cid, _ds(sid * ZPW, ZPW)])


@functools.partial(
    pl.kernel,
    out_type=jax.ShapeDtypeStruct((NC, NPAD, HH), _f32),
    mesh=_mesh,
    compiler_params=_cp,
    scratch_types=[
        pltpu.VMEM((NCHUNK, CHUNK), _i32),
        pltpu.VMEM((NCHUNK, CHUNK), _i32),
        pltpu.VMEM((EPW,), _f32),
        pltpu.VMEM((2 * CHUNK, HH), _f32),
        pltpu.VMEM_SHARED((NPAD, HH), _f32),
        pltpu.SemaphoreType.DMA,
        pltpu.SemaphoreType.DMA,
    ],
)
def _kc(nm_hbm, src_hbm, dst_hbm, xw_hbm, z_hbm, acc_hbm,
        isrc, idst, nmall, xwb, acc_sh, sem_g, sem_a):
    cid = lax.axis_index("c")
    sid = lax.axis_index("s")
    wid = cid * NS + sid
    lane = lax.iota(_i32, L)

    pltpu.sync_copy(src_hbm.at[wid], isrc)
    pltpu.sync_copy(dst_hbm.at[wid], idst)
    pltpu.sync_copy(nm_hbm.at[_ds(wid * EPW, EPW)], nmall)
    pltpu.sync_copy(z_hbm, acc_sh.at[_ds(sid * ZPW, ZPW)])
    plsc.subcore_barrier()

    pltpu.async_copy(xw_hbm.at[isrc.at[0]], xwb.at[_ds(0, CHUNK)], sem_g)

    @pl.loop(0, NCHUNK)
    def _(c):
        boff = jnp.bitwise_and(c, 1) * CHUNK
        nboff = CHUNK - boff

        @pl.when(c >= 1)
        def _():
            pltpu.make_async_copy(xwb.at[_ds(nboff, CHUNK)],
                                  acc_sh.at[idst.at[c - 1]], sem_a).wait()

        @pl.when(c + 1 < NCHUNK)
        def _():
            pltpu.async_copy(xw_hbm.at[isrc.at[c + 1]],
                             xwb.at[_ds(nboff, CHUNK)], sem_g)

        pltpu.make_async_copy(xw_hbm.at[isrc.at[c]],
                              xwb.at[_ds(boff, CHUNK)], sem_g).wait()

        @pl.loop(0, CHUNK // 4)
        def _(e4):
            e0 = e4 * 4
            for t in range(4):
                rowv = jnp.full((L,), 0, _i32) + (boff + e0 + t)
                nb = plsc.load_gather(nmall, [jnp.full((L,), 0, _i32)
                                              + (c * CHUNK + e0 + t)])
                for k in range(HH // L):
                    colv = lane + k * L
                    v = plsc.load_gather(xwb, [rowv, colv])
                    plsc.store_scatter(xwb, [rowv, colv], v * nb)

        pltpu.async_copy(xwb.at[_ds(boff, CHUNK)],
                         acc_sh.at[idst.at[c]], sem_a, add=True)

    pltpu.make_async_copy(xwb.at[_ds(0, CHUNK)],
                          acc_sh.at[idst.at[NCHUNK - 1]], sem_a).wait()

    plsc.subcore_barrier()
    pltpu.sync_copy(acc_sh.at[_ds(sid * ZPW, ZPW)],
                    acc_hbm.at[cid, _ds(sid * ZPW, ZPW)])


def _m2_body(acc_ref, b1_ref, w2_ref, out_ref):
    a = acc_ref[0] + acc_ref[1]
    x1 = jnp.maximum(a + b1_ref[...], 0.0)
    out_ref[...] = jnp.dot(x1, w2_ref[...], preferred_element_type=_f32)


def _m2(acc1, b1, W2pad):
    return pl.pallas_call(
        _m2_body,
        grid=(NN // RB,),
        in_specs=[
            pl.BlockSpec((2, RB, HH), lambda i: (0, i, 0)),
            pl.BlockSpec((1, HH), lambda i: (0, 0)),
            pl.BlockSpec((HH, HH), lambda i: (0, 0)),
        ],
        out_specs=pl.BlockSpec((RB, HH), lambda i: (i, 0)),
        out_shape=jax.ShapeDtypeStruct((NN, HH), _f32),
    )(acc1, b1, W2pad)


def _m3_body(acc_ref, b2_ref, out_ref):
    out_ref[...] = acc_ref[0, :, :CC] + acc_ref[1, :, :CC] + b2_ref[...]


def _m3(acc2, b2):
    return pl.pallas_call(
        _m3_body,
        grid=(NN // RB,),
        in_specs=[
            pl.BlockSpec((2, RB, HH), lambda i: (0, i, 0)),
            pl.BlockSpec((1, CC), lambda i: (0, 0)),
        ],
        out_specs=pl.BlockSpec((RB, CC), lambda i: (i, 0)),
        out_shape=jax.ShapeDtypeStruct((NN, CC), _f32),
    )(acc2, b2)


def kernel(inputs, edge_index, W_gl, a_gl, W1, b1, W2, b2):
    src3 = edge_index[0].reshape(NW, NCHUNK, CHUNK)
    dst3 = edge_index[1].reshape(NW, NCHUNK, CHUNK)
    rot = (jnp.arange(HH)[:, None] + jnp.arange(L)[None, :]) % HH
    a_tiled = a_gl[rot].reshape(HH * L)
    W2pad = jnp.zeros((HH, HH), _f32).at[:, :CC].set(W2)
    zblk = jnp.zeros((ZPW, HH), _f32)

    h, xw1 = _m1(inputs, W_gl, W1)
    ex, s_parts = _ka(h, src3, dst3, a_tiled)
    rec, ind = _ms(s_parts)
    ew, norm, acc1 = _kb(rec, ind, ex, src3, dst3, xw1, zblk)
    xw2 = _m2(acc1, b1.reshape(1, HH), W2pad)
    acc2 = _kc(norm, src3, dst3, xw2, zblk)
    x2 = _m3(acc2, b2.reshape(1, CC))
    return (h, ew, x2)


# speedup vs baseline: 1.6708x; 1.0333x over previous
import dataclasses
import functools

import jax
import jax.numpy as jnp
from jax import lax
from jax.experimental import pallas as pl
from jax.experimental.pallas import tpu as pltpu
from jax.experimental.pallas import tpu_sc as plsc

NN = 10000
EE = 320000
FF = 128
HH = 64
CC = 40
NC, NS, L = 2, 16, 16
NW = NC * NS
NPAD = 10240
CHUNK = 80
EPW = EE // NW
NCHUNK = EPW // CHUNK
RB = 1000
ZPW = NPAD // NS

_mesh = plsc.VectorSubcoreMesh(core_axis_name="c", subcore_axis_name="s")
_cp = pltpu.CompilerParams()
if "needs_layout_passes" in pltpu.CompilerParams.__dataclass_fields__:
    _cp = dataclasses.replace(_cp, needs_layout_passes=False)
if "use_tc_tiling_on_sc" in pltpu.CompilerParams.__dataclass_fields__:
    _cp = dataclasses.replace(_cp, use_tc_tiling_on_sc=False)
_f32 = jnp.float32
_i32 = jnp.int32


def _ds(b, n):
    return pl.ds(b, n)


def _m1_body(x_ref, wg_ref, w1_ref, h_ref, xw_ref):
    x = x_ref[...]
    h_ref[...] = jnp.dot(x, wg_ref[...], preferred_element_type=_f32)
    xw_ref[...] = jnp.dot(x, w1_ref[...], preferred_element_type=_f32)


def _m1(inputs, W_gl, W1):
    return pl.pallas_call(
        _m1_body,
        grid=(NN // RB,),
        in_specs=[
            pl.BlockSpec((RB, FF), lambda i: (i, 0)),
            pl.BlockSpec((FF, HH), lambda i: (0, 0)),
            pl.BlockSpec((FF, HH), lambda i: (0, 0)),
        ],
        out_specs=[
            pl.BlockSpec((RB, HH), lambda i: (i, 0)),
            pl.BlockSpec((RB, HH), lambda i: (i, 0)),
        ],
        out_shape=[
            jax.ShapeDtypeStruct((NN, HH), _f32),
            jax.ShapeDtypeStruct((NN, HH), _f32),
        ],
    )(inputs, W_gl, W1)


@functools.partial(
    pl.kernel,
    out_type=(
        jax.ShapeDtypeStruct((EE,), _f32),
        jax.ShapeDtypeStruct((NW, NPAD), _f32),
    ),
    mesh=_mesh,
    compiler_params=_cp,
    scratch_types=[
        pltpu.VMEM((NPAD,), _f32),
        pltpu.VMEM((NCHUNK, CHUNK), _i32),
        pltpu.VMEM((NCHUNK, CHUNK), _i32),
        pltpu.VMEM((2 * CHUNK, HH), _f32),
        pltpu.VMEM((2 * CHUNK, HH), _f32),
        pltpu.VMEM((EPW,), _f32),
        pltpu.VMEM((HH * L,), _f32),
        pltpu.SemaphoreType.DMA,
    ],
)
def _ka(h_hbm, src_hbm, dst_hbm, at_hbm, ex_hbm, s_hbm,
        s_acc, isrc, idst, hs, hd, exall, at_v, sem):
    cid = lax.axis_index("c")
    sid = lax.axis_index("s")
    wid = cid * NS + sid
    lane = lax.iota(_i32, L)

    pltpu.sync_copy(at_hbm, at_v)
    pltpu.sync_copy(src_hbm.at[wid], isrc)
    pltpu.sync_copy(dst_hbm.at[wid], idst)

    zv = jnp.zeros((L,), _f32)

    @pl.loop(0, NPAD // L)
    def _(i):
        s_acc[_ds(i * L, L)] = zv

    pltpu.async_copy(h_hbm.at[isrc.at[0]], hs.at[_ds(0, CHUNK)], sem)
    pltpu.async_copy(h_hbm.at[idst.at[0]], hd.at[_ds(0, CHUNK)], sem)

    @pl.loop(0, NCHUNK)
    def _(c):
        boff = jnp.bitwise_and(c, 1) * CHUNK
        nboff = CHUNK - boff

        @pl.when(c + 1 < NCHUNK)
        def _():
            pltpu.async_copy(h_hbm.at[isrc.at[c + 1]],
                             hs.at[_ds(nboff, CHUNK)], sem)
            pltpu.async_copy(h_hbm.at[idst.at[c + 1]],
                             hd.at[_ds(nboff, CHUNK)], sem)

        pltpu.make_async_copy(h_hbm.at[isrc.at[c]],
                              hs.at[_ds(boff, CHUNK)], sem).wait()
        pltpu.make_async_copy(h_hbm.at[idst.at[c]],
                              hd.at[_ds(boff, CHUNK)], sem).wait()

        cv = jnp.full((L,), 0, _i32) + c

        @pl.loop(0, CHUNK // L)
        def _(g):
            rowg = lane + g * L
            browg = rowg + boff
            accs = [jnp.zeros((L,), _f32) for _ in range(4)]
            for j in range(HH):
                colj = jnp.bitwise_and(lane + j, HH - 1)
                hsj = plsc.load_gather(hs, [browg, colj])
                hdj = plsc.load_gather(hd, [browg, colj])
                aj = at_v[_ds(j * L, L)]
                accs[j % 4] = accs[j % 4] + aj * jnp.abs(hsj - hdj)
            acc = (accs[0] + accs[1]) + (accs[2] + accs[3])
            exv = jnp.exp(jnp.maximum(acc, 0.0))
            dstv = plsc.load_gather(idst, [cv, rowg])
            plsc.addupdate_scatter(s_acc, [dstv], exv)
            exall[_ds(c * CHUNK + g * L, L)] = exv

    pltpu.sync_copy(exall, ex_hbm.at[_ds(wid * EPW, EPW)])
    pltpu.sync_copy(s_acc, s_hbm.at[wid])


def _ms_body(sp_ref, rec_ref, ind_ref):
    s = jnp.sum(sp_ref[...], axis=0).reshape(1, NPAD)
    pos = s > 0.0
    rec_ref[...] = jnp.where(pos, 1.0 / s, 0.0)
    ind_ref[...] = jnp.where(pos, 1.0, 0.0)


def _ms(s_parts):
    return pl.pallas_call(
        _ms_body,
        grid=(1,),
        in_specs=[pl.BlockSpec((NW, NPAD), lambda i: (0, 0))],
        out_specs=[
            pl.BlockSpec((1, NPAD), lambda i: (0, 0)),
            pl.BlockSpec((1, NPAD), lambda i: (0, 0)),
        ],
        out_shape=[
            jax.ShapeDtypeStruct((1, NPAD), _f32),
            jax.ShapeDtypeStruct((1, NPAD), _f32),
        ],
    )(s_parts)


@functools.partial(
    pl.kernel,
    out_type=(
        jax.ShapeDtypeStruct((EE,), _f32),
        jax.ShapeDtypeStruct((EE,), _f32),
        jax.ShapeDtypeStruct((NC, NPAD, HH), _f32),
    ),
    mesh=_mesh,
    compiler_params=_cp,
    scratch_types=[
        pltpu.VMEM((NPAD,), _f32),
        pltpu.VMEM((NPAD,), _f32),
        pltpu.VMEM((NCHUNK, CHUNK), _i32),
        pltpu.VMEM((NCHUNK, CHUNK), _i32),
        pltpu.VMEM((EPW,), _f32),
        pltpu.VMEM((EPW,), _f32),
        pltpu.VMEM((EPW,), _f32),
        pltpu.VMEM((2 * CHUNK, HH), _f32),
        pltpu.VMEM_SHARED((NPAD, HH), _f32),
        pltpu.SemaphoreType.DMA,
        pltpu.SemaphoreType.DMA,
    ],
)
def _kb(rec_hbm, ind_hbm, ex_hbm, src_hbm, dst_hbm, xw_hbm, z_hbm,
        ew_hbm, nm_hbm, acc_hbm,
        rec, ind, isrc, idst, exall, ewall, nmall, xwb, acc_sh, sem_g, sem_a):
    cid = lax.axis_index("c")
    sid = lax.axis_index("s")
    wid = cid * NS + sid
    lane = lax.iota(_i32, L)

    pltpu.sync_copy(rec_hbm.at[0], rec)
    pltpu.sync_copy(ind_hbm.at[0], ind)
    pltpu.sync_copy(src_hbm.at[wid], isrc)
    pltpu.sync_copy(dst_hbm.at[wid], idst)
    pltpu.sync_copy(ex_hbm.at[_ds(wid * EPW, EPW)], exall)
    pltpu.sync_copy(z_hbm, acc_sh.at[_ds(sid * ZPW, ZPW)])
    plsc.subcore_barrier()

    @pl.loop(0, NCHUNK)
    def _(c):
        cv = jnp.full((L,), 0, _i32) + c
        for g in range(CHUNK // L):
            colg = lane + g * L
            srcv = plsc.load_gather(isrc, [cv, colg])
            dstv = plsc.load_gather(idst, [cv, colg])
            exv = exall[_ds(c * CHUNK + g * L, L)]
            ew = exv * plsc.load_gather(rec, [dstv])
            nm = ew * plsc.load_gather(ind, [srcv])
            ewall[_ds(c * CHUNK + g * L, L)] = ew
            nmall[_ds(c * CHUNK + g * L, L)] = nm

    pltpu.async_copy(xw_hbm.at[isrc.at[0]], xwb.at[_ds(0, CHUNK)], sem_g)

    @pl.loop(0, NCHUNK)
    def _(c):
        boff = jnp.bitwise_and(c, 1) * CHUNK
        nboff = CHUNK - boff

        @pl.when(c >= 1)
        def _():
            pltpu.make_async_copy(xwb.at[_ds(nboff, CHUNK)],
                                  acc_sh.at[idst.at[c - 1]], sem_a).wait()

        @pl.when(c + 1 < NCHUNK)
        def _():
            pltpu.async_copy(xw_hbm.at[isrc.at[c + 1]],
                             xwb.at[_ds(nboff, CHUNK)], sem_g)

        pltpu.make_async_copy(xw_hbm.at[isrc.at[c]],
                              xwb.at[_ds(boff, CHUNK)], sem_g).wait()

        @pl.loop(0, CHUNK // L)
        def _(g):
            nmv = nmall[_ds(c * CHUNK + g * L, L)]
            rowv = boff + g * L + lane
            for j in range(HH):
                colj = jnp.bitwise_and(lane + j, HH - 1)
                v = plsc.load_gather(xwb, [rowv, colj])
                plsc.store_scatter(xwb, [rowv, colj], v * nmv)

        pltpu.async_copy(xwb.at[_ds(boff, CHUNK)],
                         acc_sh.at[idst.at[c]], sem_a, add=True)

    pltpu.make_async_copy(xwb.at[_ds(0, CHUNK)],
                          acc_sh.at[idst.at[NCHUNK - 1]], sem_a).wait()
    pltpu.sync_copy(ewall, ew_hbm.at[_ds(wid * EPW, EPW)])
    pltpu.sync_copy(nmall, nm_hbm.at[_ds(wid * EPW, EPW)])

    plsc.subcore_barrier()
    pltpu.sync_copy(acc_sh.at[_ds(sid * ZPW, ZPW)],
                    acc_hbm.at[cid, _ds(sid * ZPW, ZPW)])


@functools.partial(
    pl.kernel,
    out_type=jax.ShapeDtypeStruct((NC, NPAD, HH), _f32),
    mesh=_mesh,
    compiler_params=_cp,
    scratch_types=[
        pltpu.VMEM((NCHUNK, CHUNK), _i32),
        pltpu.VMEM((NCHUNK, CHUNK), _i32),
        pltpu.VMEM((EPW,), _f32),
        pltpu.VMEM((2 * CHUNK, HH), _f32),
        pltpu.VMEM_SHARED((NPAD, HH), _f32),
        pltpu.SemaphoreType.DMA,
        pltpu.SemaphoreType.DMA,
    ],
)
def _kc(nm_hbm, src_hbm, dst_hbm, xw_hbm, z_hbm, acc_hbm,
        isrc, idst, nmall, xwb, acc_sh, sem_g, sem_a):
    cid = lax.axis_index("c")
    sid = lax.axis_index("s")
    wid = cid * NS + sid
    lane = lax.iota(_i32, L)

    pltpu.sync_copy(src_hbm.at[wid], isrc)
    pltpu.sync_copy(dst_hbm.at[wid], idst)
    pltpu.sync_copy(nm_hbm.at[_ds(wid * EPW, EPW)], nmall)
    pltpu.sync_copy(z_hbm, acc_sh.at[_ds(sid * ZPW, ZPW)])
    plsc.subcore_barrier()

    pltpu.async_copy(xw_hbm.at[isrc.at[0]], xwb.at[_ds(0, CHUNK)], sem_g)

    @pl.loop(0, NCHUNK)
    def _(c):
        boff = jnp.bitwise_and(c, 1) * CHUNK
        nboff = CHUNK - boff

        @pl.when(c >= 1)
        def _():
            pltpu.make_async_copy(xwb.at[_ds(nboff, CHUNK)],
                                  acc_sh.at[idst.at[c - 1]], sem_a).wait()

        @pl.when(c + 1 < NCHUNK)
        def _():
            pltpu.async_copy(xw_hbm.at[isrc.at[c + 1]],
                             xwb.at[_ds(nboff, CHUNK)], sem_g)

        pltpu.make_async_copy(xw_hbm.at[isrc.at[c]],
                              xwb.at[_ds(boff, CHUNK)], sem_g).wait()

        @pl.loop(0, CHUNK // L)
        def _(g):
            nmv = nmall[_ds(c * CHUNK + g * L, L)]
            rowv = boff + g * L + lane
            for j in range(HH):
                colj = jnp.bitwise_and(lane + j, HH - 1)
                v = plsc.load_gather(xwb, [rowv, colj])
                plsc.store_scatter(xwb, [rowv, colj], v * nmv)

        pltpu.async_copy(xwb.at[_ds(boff, CHUNK)],
                         acc_sh.at[idst.at[c]], sem_a, add=True)

    pltpu.make_async_copy(xwb.at[_ds(0, CHUNK)],
                          acc_sh.at[idst.at[NCHUNK - 1]], sem_a).wait()

    plsc.subcore_barrier()
    pltpu.sync_copy(acc_sh.at[_ds(sid * ZPW, ZPW)],
                    acc_hbm.at[cid, _ds(sid * ZPW, ZPW)])


def _m2_body(acc_ref, b1_ref, w2_ref, out_ref):
    a = acc_ref[0] + acc_ref[1]
    x1 = jnp.maximum(a + b1_ref[...], 0.0)
    out_ref[...] = jnp.dot(x1, w2_ref[...], preferred_element_type=_f32)


def _m2(acc1, b1, W2pad):
    return pl.pallas_call(
        _m2_body,
        grid=(NN // RB,),
        in_specs=[
            pl.BlockSpec((2, RB, HH), lambda i: (0, i, 0)),
            pl.BlockSpec((1, HH), lambda i: (0, 0)),
            pl.BlockSpec((HH, HH), lambda i: (0, 0)),
        ],
        out_specs=pl.BlockSpec((RB, HH), lambda i: (i, 0)),
        out_shape=jax.ShapeDtypeStruct((NN, HH), _f32),
    )(acc1, b1, W2pad)


def _m3_body(acc_ref, b2_ref, out_ref):
    out_ref[...] = acc_ref[0, :, :CC] + acc_ref[1, :, :CC] + b2_ref[...]


def _m3(acc2, b2):
    return pl.pallas_call(
        _m3_body,
        grid=(NN // RB,),
        in_specs=[
            pl.BlockSpec((2, RB, HH), lambda i: (0, i, 0)),
            pl.BlockSpec((1, CC), lambda i: (0, 0)),
        ],
        out_specs=pl.BlockSpec((RB, CC), lambda i: (i, 0)),
        out_shape=jax.ShapeDtypeStruct((NN, CC), _f32),
    )(acc2, b2)


def kernel(inputs, edge_index, W_gl, a_gl, W1, b1, W2, b2):
    src3 = edge_index[0].reshape(NW, NCHUNK, CHUNK)
    dst3 = edge_index[1].reshape(NW, NCHUNK, CHUNK)
    rot = (jnp.arange(HH)[:, None] + jnp.arange(L)[None, :]) % HH
    a_tiled = a_gl[rot].reshape(HH * L)
    W2pad = jnp.zeros((HH, HH), _f32).at[:, :CC].set(W2)
    zblk = jnp.zeros((ZPW, HH), _f32)

    h, xw1 = _m1(inputs, W_gl, W1)
    ex, s_parts = _ka(h, src3, dst3, a_tiled)
    rec, ind = _ms(s_parts)
    ew, norm, acc1 = _kb(rec, ind, ex, src3, dst3, xw1, zblk)
    xw2 = _m2(acc1, b1.reshape(1, HH), W2pad)
    acc2 = _kc(norm, src3, dst3, xw2, zblk)
    x2 = _m3(acc2, b2.reshape(1, CC))
    return (h, ew, x2)


# speedup vs baseline: 1.8111x; 1.0839x over previous
import dataclasses
import functools

import jax
import jax.numpy as jnp
from jax import lax
from jax.experimental import pallas as pl
from jax.experimental.pallas import tpu as pltpu
from jax.experimental.pallas import tpu_sc as plsc

NN = 10000
EE = 320000
FF = 128
HH = 64
CC = 40
NC, NS, L = 2, 16, 16
NW = NC * NS
NPAD = 10240
CHUNK = 80
EPW = EE // NW
NCHUNK = EPW // CHUNK
RB = 1000
ZPW = NPAD // NS

_mesh = plsc.VectorSubcoreMesh(core_axis_name="c", subcore_axis_name="s")
_cp = pltpu.CompilerParams()
if "needs_layout_passes" in pltpu.CompilerParams.__dataclass_fields__:
    _cp = dataclasses.replace(_cp, needs_layout_passes=False)
if "use_tc_tiling_on_sc" in pltpu.CompilerParams.__dataclass_fields__:
    _cp = dataclasses.replace(_cp, use_tc_tiling_on_sc=False)
_f32 = jnp.float32
_i32 = jnp.int32


def _ds(b, n):
    return pl.ds(b, n)


def _m1_body(x_ref, wg_ref, w1_ref, h_ref, xw_ref):
    x = x_ref[...]
    h_ref[...] = jnp.dot(x, wg_ref[...], preferred_element_type=_f32)
    xw_ref[...] = jnp.dot(x, w1_ref[...], preferred_element_type=_f32)


def _m1(inputs, W_gl, W1):
    return pl.pallas_call(
        _m1_body,
        grid=(NN // RB,),
        in_specs=[
            pl.BlockSpec((RB, FF), lambda i: (i, 0)),
            pl.BlockSpec((FF, HH), lambda i: (0, 0)),
            pl.BlockSpec((FF, HH), lambda i: (0, 0)),
        ],
        out_specs=[
            pl.BlockSpec((RB, HH), lambda i: (i, 0)),
            pl.BlockSpec((RB, HH), lambda i: (i, 0)),
        ],
        out_shape=[
            jax.ShapeDtypeStruct((NN, HH), _f32),
            jax.ShapeDtypeStruct((NN, HH), _f32),
        ],
    )(inputs, W_gl, W1)


@functools.partial(
    pl.kernel,
    out_type=(
        jax.ShapeDtypeStruct((EE,), _f32),
        jax.ShapeDtypeStruct((NW, NPAD), _f32),
    ),
    mesh=_mesh,
    compiler_params=_cp,
    scratch_types=[
        pltpu.VMEM((NPAD,), _f32),
        pltpu.VMEM((NCHUNK, CHUNK), _i32),
        pltpu.VMEM((NCHUNK, CHUNK), _i32),
        pltpu.VMEM((2 * CHUNK, HH), _f32),
        pltpu.VMEM((2 * CHUNK, HH), _f32),
        pltpu.VMEM((EPW,), _f32),
        pltpu.VMEM((HH * L,), _f32),
        pltpu.SemaphoreType.DMA,
    ],
)
def _ka(h_hbm, src_hbm, dst_hbm, at_hbm, ex_hbm, s_hbm,
        s_acc, isrc, idst, hs, hd, exall, at_v, sem):
    cid = lax.axis_index("c")
    sid = lax.axis_index("s")
    wid = cid * NS + sid
    lane = lax.iota(_i32, L)

    pltpu.sync_copy(at_hbm, at_v)
    pltpu.sync_copy(src_hbm.at[wid], isrc)
    pltpu.sync_copy(dst_hbm.at[wid], idst)

    zv = jnp.zeros((L,), _f32)

    @pl.loop(0, NPAD // L)
    def _(i):
        s_acc[_ds(i * L, L)] = zv

    pltpu.async_copy(h_hbm.at[isrc.at[0]], hs.at[_ds(0, CHUNK)], sem)
    pltpu.async_copy(h_hbm.at[idst.at[0]], hd.at[_ds(0, CHUNK)], sem)

    @pl.loop(0, NCHUNK)
    def _(c):
        boff = jnp.bitwise_and(c, 1) * CHUNK
        nboff = CHUNK - boff

        @pl.when(c + 1 < NCHUNK)
        def _():
            pltpu.async_copy(h_hbm.at[isrc.at[c + 1]],
                             hs.at[_ds(nboff, CHUNK)], sem)
            pltpu.async_copy(h_hbm.at[idst.at[c + 1]],
                             hd.at[_ds(nboff, CHUNK)], sem)

        pltpu.make_async_copy(h_hbm.at[isrc.at[c]],
                              hs.at[_ds(boff, CHUNK)], sem).wait()
        pltpu.make_async_copy(h_hbm.at[idst.at[c]],
                              hd.at[_ds(boff, CHUNK)], sem).wait()

        cv = jnp.full((L,), 0, _i32) + c

        @pl.loop(0, CHUNK // L)
        def _(g):
            rowg = lane + g * L
            browg = rowg + boff
            accs = [jnp.zeros((L,), _f32) for _ in range(4)]
            for j in range(HH):
                colj = jnp.bitwise_and(lane + j, HH - 1)
                hsj = plsc.load_gather(hs, [browg, colj])
                hdj = plsc.load_gather(hd, [browg, colj])
                aj = at_v[_ds(j * L, L)]
                accs[j % 4] = accs[j % 4] + aj * jnp.abs(hsj - hdj)
            acc = (accs[0] + accs[1]) + (accs[2] + accs[3])
            exv = jnp.exp(jnp.maximum(acc, 0.0))
            dstv = plsc.load_gather(idst, [cv, rowg])
            plsc.addupdate_scatter(s_acc, [dstv], exv)
            exall[_ds(c * CHUNK + g * L, L)] = exv

    pltpu.sync_copy(exall, ex_hbm.at[_ds(wid * EPW, EPW)])
    pltpu.sync_copy(s_acc, s_hbm.at[wid])


def _ms_body(sp_ref, rec_ref, ind_ref):
    s = jnp.sum(sp_ref[...], axis=0).reshape(1, NPAD)
    pos = s > 0.0
    rec_ref[...] = jnp.where(pos, 1.0 / s, 0.0)
    ind_ref[...] = jnp.where(pos, 1.0, 0.0)


def _ms(s_parts):
    return pl.pallas_call(
        _ms_body,
        grid=(1,),
        in_specs=[pl.BlockSpec((NW, NPAD), lambda i: (0, 0))],
        out_specs=[
            pl.BlockSpec((1, NPAD), lambda i: (0, 0)),
            pl.BlockSpec((1, NPAD), lambda i: (0, 0)),
        ],
        out_shape=[
            jax.ShapeDtypeStruct((1, NPAD), _f32),
            jax.ShapeDtypeStruct((1, NPAD), _f32),
        ],
    )(s_parts)


@functools.partial(
    pl.kernel,
    out_type=(
        jax.ShapeDtypeStruct((EE,), _f32),
        jax.ShapeDtypeStruct((EE,), _f32),
        jax.ShapeDtypeStruct((NC, NPAD, HH), _f32),
    ),
    mesh=_mesh,
    compiler_params=_cp,
    scratch_types=[
        pltpu.VMEM((NPAD,), _f32),
        pltpu.VMEM((NPAD,), _f32),
        pltpu.VMEM((NCHUNK, CHUNK), _i32),
        pltpu.VMEM((NCHUNK, CHUNK), _i32),
        pltpu.VMEM((EPW,), _f32),
        pltpu.VMEM((EPW,), _f32),
        pltpu.VMEM((EPW,), _f32),
        pltpu.VMEM((3 * CHUNK, HH), _f32),
        pltpu.VMEM_SHARED((NPAD, HH), _f32),
        pltpu.SemaphoreType.DMA,
        pltpu.SemaphoreType.DMA,
    ],
)
def _kb(rec_hbm, ind_hbm, ex_hbm, src_hbm, dst_hbm, xw_hbm, z_hbm,
        ew_hbm, nm_hbm, acc_hbm,
        rec, ind, isrc, idst, exall, ewall, nmall, xwb, acc_sh, sem_g, sem_a):
    cid = lax.axis_index("c")
    sid = lax.axis_index("s")
    wid = cid * NS + sid
    lane = lax.iota(_i32, L)

    pltpu.sync_copy(rec_hbm.at[0], rec)
    pltpu.sync_copy(ind_hbm.at[0], ind)
    pltpu.sync_copy(src_hbm.at[wid], isrc)
    pltpu.sync_copy(dst_hbm.at[wid], idst)
    pltpu.sync_copy(ex_hbm.at[_ds(wid * EPW, EPW)], exall)
    pltpu.sync_copy(z_hbm, acc_sh.at[_ds(sid * ZPW, ZPW)])
    plsc.subcore_barrier()

    @pl.loop(0, NCHUNK)
    def _(c):
        cv = jnp.full((L,), 0, _i32) + c
        for g in range(CHUNK // L):
            colg = lane + g * L
            srcv = plsc.load_gather(isrc, [cv, colg])
            dstv = plsc.load_gather(idst, [cv, colg])
            exv = exall[_ds(c * CHUNK + g * L, L)]
            ew = exv * plsc.load_gather(rec, [dstv])
            nm = ew * plsc.load_gather(ind, [srcv])
            ewall[_ds(c * CHUNK + g * L, L)] = ew
            nmall[_ds(c * CHUNK + g * L, L)] = nm

    pltpu.async_copy(xw_hbm.at[isrc.at[0]], xwb.at[_ds(0, CHUNK)], sem_g)

    @pl.loop(0, NCHUNK)
    def _(c):
        boff = jnp.remainder(c, 3) * CHUNK
        nboff = jnp.remainder(c + 1, 3) * CHUNK

        @pl.when(c >= 2)
        def _():
            pltpu.make_async_copy(xwb.at[_ds(nboff, CHUNK)],
                                  acc_sh.at[idst.at[c - 2]], sem_a).wait()

        @pl.when(c + 1 < NCHUNK)
        def _():
            pltpu.async_copy(xw_hbm.at[isrc.at[c + 1]],
                             xwb.at[_ds(nboff, CHUNK)], sem_g)

        pltpu.make_async_copy(xw_hbm.at[isrc.at[c]],
                              xwb.at[_ds(boff, CHUNK)], sem_g).wait()

        @pl.loop(0, CHUNK // L)
        def _(g):
            nmv = nmall[_ds(c * CHUNK + g * L, L)]
            rowv = boff + g * L + lane
            for j in range(HH):
                colj = jnp.bitwise_and(lane + j, HH - 1)
                v = plsc.load_gather(xwb, [rowv, colj])
                plsc.store_scatter(xwb, [rowv, colj], v * nmv)

        pltpu.async_copy(xwb.at[_ds(boff, CHUNK)],
                         acc_sh.at[idst.at[c]], sem_a, add=True)

    pltpu.make_async_copy(xwb.at[_ds(0, CHUNK)],
                          acc_sh.at[idst.at[NCHUNK - 2]], sem_a).wait()
    pltpu.make_async_copy(xwb.at[_ds(0, CHUNK)],
                          acc_sh.at[idst.at[NCHUNK - 1]], sem_a).wait()
    pltpu.sync_copy(ewall, ew_hbm.at[_ds(wid * EPW, EPW)])
    pltpu.sync_copy(nmall, nm_hbm.at[_ds(wid * EPW, EPW)])

    plsc.subcore_barrier()
    pltpu.sync_copy(acc_sh.at[_ds(sid * ZPW, ZPW)],
                    acc_hbm.at[cid, _ds(sid * ZPW, ZPW)])


@functools.partial(
    pl.kernel,
    out_type=jax.ShapeDtypeStruct((NC, NPAD, HH), _f32),
    mesh=_mesh,
    compiler_params=_cp,
    scratch_types=[
        pltpu.VMEM((NCHUNK, CHUNK), _i32),
        pltpu.VMEM((NCHUNK, CHUNK), _i32),
        pltpu.VMEM((EPW,), _f32),
        pltpu.VMEM((2 * CHUNK, HH), _f32),
        pltpu.VMEM_SHARED((NPAD, HH), _f32),
        pltpu.SemaphoreType.DMA,
        pltpu.SemaphoreType.DMA,
    ],
)
def _kc(nm_hbm, src_hbm, dst_hbm, xw_hbm, z_hbm, acc_hbm,
        isrc, idst, nmall, xwb, acc_sh, sem_g, sem_a):
    cid = lax.axis_index("c")
    sid = lax.axis_index("s")
    wid = cid * NS + sid
    lane = lax.iota(_i32, L)

    pltpu.sync_copy(src_hbm.at[wid], isrc)
    pltpu.sync_copy(dst_hbm.at[wid], idst)
    pltpu.sync_copy(nm_hbm.at[_ds(wid * EPW, EPW)], nmall)
    pltpu.sync_copy(z_hbm, acc_sh.at[_ds(sid * ZPW, ZPW)])
    plsc.subcore_barrier()

    pltpu.async_copy(xw_hbm.at[isrc.at[0]], xwb.at[_ds(0, CHUNK)], sem_g)

    @pl.loop(0, NCHUNK)
    def _(c):
        boff = jnp.remainder(c, 3) * CHUNK
        nboff = jnp.remainder(c + 1, 3) * CHUNK

        @pl.when(c >= 2)
        def _():
            pltpu.make_async_copy(xwb.at[_ds(nboff, CHUNK)],
                                  acc_sh.at[idst.at[c - 2]], sem_a).wait()

        @pl.when(c + 1 < NCHUNK)
        def _():
            pltpu.async_copy(xw_hbm.at[isrc.at[c + 1]],
                             xwb.at[_ds(nboff, CHUNK)], sem_g)

        pltpu.make_async_copy(xw_hbm.at[isrc.at[c]],
                              xwb.at[_ds(boff, CHUNK)], sem_g).wait()

        @pl.loop(0, CHUNK // L)
        def _(g):
            nmv = nmall[_ds(c * CHUNK + g * L, L)]
            rowv = boff + g * L + lane
            for j in range(HH):
                colj = jnp.bitwise_and(lane + j, HH - 1)
                v = plsc.load_gather(xwb, [rowv, colj])
                plsc.store_scatter(xwb, [rowv, colj], v * nmv)

        pltpu.async_copy(xwb.at[_ds(boff, CHUNK)],
                         acc_sh.at[idst.at[c]], sem_a, add=True)

    pltpu.make_async_copy(xwb.at[_ds(0, CHUNK)],
                          acc_sh.at[idst.at[NCHUNK - 2]], sem_a).wait()
    pltpu.make_async_copy(xwb.at[_ds(0, CHUNK)],
                          acc_sh.at[idst.at[NCHUNK - 1]], sem_a).wait()

    plsc.subcore_barrier()
    pltpu.sync_copy(acc_sh.at[_ds(sid * ZPW, ZPW)],
                    acc_hbm.at[cid, _ds(sid * ZPW, ZPW)])


def _m2_body(acc_ref, b1_ref, w2_ref, out_ref):
    a = acc_ref[0] + acc_ref[1]
    x1 = jnp.maximum(a + b1_ref[...], 0.0)
    out_ref[...] = jnp.dot(x1, w2_ref[...], preferred_element_type=_f32)


def _m2(acc1, b1, W2pad):
    return pl.pallas_call(
        _m2_body,
        grid=(NN // RB,),
        in_specs=[
            pl.BlockSpec((2, RB, HH), lambda i: (0, i, 0)),
            pl.BlockSpec((1, HH), lambda i: (0, 0)),
            pl.BlockSpec((HH, HH), lambda i: (0, 0)),
        ],
        out_specs=pl.BlockSpec((RB, HH), lambda i: (i, 0)),
        out_shape=jax.ShapeDtypeStruct((NN, HH), _f32),
    )(acc1, b1, W2pad)


def _m3_body(acc_ref, b2_ref, out_ref):
    out_ref[...] = acc_ref[0, :, :CC] + acc_ref[1, :, :CC] + b2_ref[...]


def _m3(acc2, b2):
    return pl.pallas_call(
        _m3_body,
        grid=(NN // RB,),
        in_specs=[
            pl.BlockSpec((2, RB, HH), lambda i: (0, i, 0)),
            pl.BlockSpec((1, CC), lambda i: (0, 0)),
        ],
        out_specs=pl.BlockSpec((RB, CC), lambda i: (i, 0)),
        out_shape=jax.ShapeDtypeStruct((NN, CC), _f32),
    )(acc2, b2)


def kernel(inputs, edge_index, W_gl, a_gl, W1, b1, W2, b2):
    src3 = edge_index[0].reshape(NW, NCHUNK, CHUNK)
    dst3 = edge_index[1].reshape(NW, NCHUNK, CHUNK)
    rot = (jnp.arange(HH)[:, None] + jnp.arange(L)[None, :]) % HH
    a_tiled = a_gl[rot].reshape(HH * L)
    W2pad = jnp.zeros((HH, HH), _f32).at[:, :CC].set(W2)
    zblk = jnp.zeros((ZPW, HH), _f32)

    h, xw1 = _m1(inputs, W_gl, W1)
    ex, s_parts = _ka(h, src3, dst3, a_tiled)
    rec, ind = _ms(s_parts)
    ew, norm, acc1 = _kb(rec, ind, ex, src3, dst3, xw1, zblk)
    xw2 = _m2(acc1, b1.reshape(1, HH), W2pad)
    acc2 = _kc(norm, src3, dst3, xw2, zblk)
    x2 = _m3(acc2, b2.reshape(1, CC))
    return (h, ew, x2)
